# Initial kernel scaffold; baseline (speedup 1.0000x reference)
import functools

import jax
import jax.numpy as jnp
from jax import lax
from jax.experimental import pallas as pl
from jax.experimental.pallas import tpu as pltpu
from jax.experimental.pallas import tpu_sc as plsc

N_NODES = 10000
D = 128
N_EDGES = 320000

NC = 2
NS = 16
NW = NC * NS

RPT = 640
NPAD = NS * RPT
BLK = 128
NB = 79
EPT = NB * BLK
EPAD = NW * EPT

_mesh = plsc.VectorSubcoreMesh(core_axis_name="c", subcore_axis_name="s")


@functools.partial(
    pl.kernel,
    mesh=_mesh,
    out_type=jax.ShapeDtypeStruct((NC, NPAD, 8), jnp.float32),
    scratch_types=[
        pltpu.VMEM_SHARED((NPAD, 8), jnp.float32),
        pltpu.VMEM((RPT, 8), jnp.float32),
        pltpu.VMEM((128, 8), jnp.float32),
        pltpu.VMEM((NB, BLK), jnp.int32),
    ],
)
def _deg_kernel(dst_hbm, ones_hbm, zeros_hbm, out_hbm, acc, zbuf, ones_v, idx_v):
    c = lax.axis_index("c")
    s = lax.axis_index("s")
    base = s * RPT
    pltpu.sync_copy(zeros_hbm, zbuf)
    pltpu.sync_copy(zbuf, acc.at[pl.ds(base, RPT)])
    pltpu.sync_copy(ones_hbm, ones_v)
    pltpu.sync_copy(dst_hbm.at[c, s], idx_v)
    plsc.subcore_barrier()

    def body(j, carry):
        pltpu.sync_copy(ones_v, acc.at[idx_v.at[j]], add=True)
        return carry

    lax.fori_loop(0, NB, body, 0)
    plsc.subcore_barrier()
    pltpu.sync_copy(acc.at[pl.ds(base, RPT)], zbuf)
    pltpu.sync_copy(zbuf, out_hbm.at[c, pl.ds(base, RPT)])


@functools.partial(
    pl.kernel,
    mesh=_mesh,
    out_type=jax.ShapeDtypeStruct((NC, NPAD, D), jnp.float32),
    scratch_types=[
        pltpu.VMEM_SHARED((NPAD, D), jnp.float32),
        pltpu.VMEM((BLK, D), jnp.float32),
        pltpu.VMEM((NB, BLK), jnp.int32),
        pltpu.VMEM((NB, BLK), jnp.int32),
        pltpu.SemaphoreType.DMA,
    ],
)
def _scatter_kernel(xs_hbm, src_hbm, dst_hbm, out_hbm, acc, buf, src_v, dst_v, sem):
    c = lax.axis_index("c")
    s = lax.axis_index("s")
    base = s * RPT
    for k in range(RPT // 128):
        sl = pl.ds(base + k * 128, 128)
        pltpu.sync_copy(xs_hbm.at[sl], buf)
        pltpu.sync_copy(buf, acc.at[sl])
    pltpu.sync_copy(src_hbm.at[c, s], src_v)
    pltpu.sync_copy(dst_hbm.at[c, s], dst_v)
    plsc.subcore_barrier()

    def body(j, carry):
        pltpu.async_copy(xs_hbm.at[src_v.at[j]], buf, sem).wait()
        pltpu.sync_copy(buf, acc.at[dst_v.at[j]], add=True)
        return carry

    lax.fori_loop(0, NB, body, 0)
    plsc.subcore_barrier()
    for k in range(RPT // 128):
        sl = pl.ds(base + k * 128, 128)
        pltpu.sync_copy(acc.at[sl], buf)
        pltpu.sync_copy(buf, out_hbm.at[c, sl])


_ROWS = 256
_GRID = NPAD // _ROWS


def _dinv_of(degp_blk):
    deg = degp_blk[0, :, 0] + degp_blk[1, :, 0] + 1.0
    return lax.rsqrt(deg)


def _prep1_body(degp_ref, x_ref, w_ref, xs_ref):
    dinv = _dinv_of(degp_ref[...])
    xw = jnp.dot(x_ref[...], w_ref[...], preferred_element_type=jnp.float32)
    xs_ref[...] = xw * dinv[:, None]


def _mid_body(degp_ref, p_ref, xs_ref, b_ref, w_ref, out_ref):
    dinv = _dinv_of(degp_ref[...])
    tot = p_ref[0] + p_ref[1] - xs_ref[...]
    h = jnp.maximum(tot * dinv[:, None] + b_ref[...], 0.0)
    out_ref[...] = jnp.dot(h, w_ref[...], preferred_element_type=jnp.float32) * dinv[:, None]


def _final_body(degp_ref, p_ref, xs_ref, b_ref, out_ref):
    dinv = _dinv_of(degp_ref[...])
    tot = p_ref[0] + p_ref[1] - xs_ref[...]
    out_ref[...] = tot * dinv[:, None] + b_ref[...]


_degp_spec = pl.BlockSpec((NC, _ROWS, 8), lambda i: (0, i, 0))
_rows_spec = pl.BlockSpec((_ROWS, D), lambda i: (i, 0))
_part_spec = pl.BlockSpec((NC, _ROWS, D), lambda i: (0, i, 0))
_w_spec = pl.BlockSpec((D, D), lambda i: (0, 0))
_b_spec = pl.BlockSpec((1, D), lambda i: (0, 0))
_f32 = lambda shape: jax.ShapeDtypeStruct(shape, jnp.float32)

_prep1 = pl.pallas_call(
    _prep1_body,
    grid=(_GRID,),
    in_specs=[_degp_spec, _rows_spec, _w_spec],
    out_specs=_rows_spec,
    out_shape=_f32((NPAD, D)),
)

_mid = pl.pallas_call(
    _mid_body,
    grid=(_GRID,),
    in_specs=[_degp_spec, _part_spec, _rows_spec, _b_spec, _w_spec],
    out_specs=_rows_spec,
    out_shape=_f32((NPAD, D)),
)

_final = pl.pallas_call(
    _final_body,
    grid=(_GRID,),
    in_specs=[_degp_spec, _part_spec, _rows_spec, _b_spec],
    out_specs=_rows_spec,
    out_shape=_f32((NPAD, D)),
)


@jax.jit
def kernel(x, edge_index, W1, b1, W2, b2):
    ei = edge_index.astype(jnp.int32)
    npadrows = NPAD - N_NODES
    pad_idx = N_NODES + (jnp.arange(EPAD - N_EDGES, dtype=jnp.int32) % npadrows)
    src = jnp.concatenate([ei[0], pad_idx]).reshape(NC, NS, NB, BLK)
    dst = jnp.concatenate([ei[1], pad_idx]).reshape(NC, NS, NB, BLK)
    x_pad = jnp.pad(x, ((0, npadrows), (0, 0)))
    ones_rows = jnp.ones((128, 8), jnp.float32)
    zeros_rows = jnp.zeros((RPT, 8), jnp.float32)

    degp = _deg_kernel(dst, ones_rows, zeros_rows)
    xs1 = _prep1(degp, x_pad, W1)
    p1 = _scatter_kernel(xs1, src, dst)
    xs2 = _mid(degp, p1, xs1, b1.reshape(1, D), W2)
    p2 = _scatter_kernel(xs2, src, dst)
    out = _final(degp, p2, xs2, b2.reshape(1, D))
    return out[:N_NODES]


# speedup vs baseline: 15.4365x; 15.4365x over previous
import functools

import jax
import jax.numpy as jnp
from jax import lax
from jax.experimental import pallas as pl
from jax.experimental.pallas import tpu as pltpu
from jax.experimental.pallas import tpu_sc as plsc

N_NODES = 10000
D = 128
N_EDGES = 320000

NC = 2
NS = 16
NW = NC * NS

RPT = 640
NPAD = NS * RPT
BLK = 128
NB = 79
EPT = NB * BLK
EPAD = NW * EPT

_mesh = plsc.VectorSubcoreMesh(core_axis_name="c", subcore_axis_name="s")


@functools.partial(
    pl.kernel,
    mesh=_mesh,
    out_type=jax.ShapeDtypeStruct((NC, NPAD, D), jnp.float32),
    scratch_types=[
        pltpu.VMEM_SHARED((NPAD, D), jnp.float32),
        pltpu.VMEM((BLK, D), jnp.float32),
        pltpu.VMEM((NB, BLK), jnp.int32),
        pltpu.VMEM((NB, BLK), jnp.int32),
        pltpu.SemaphoreType.DMA,
    ],
)
def _scatter_kernel(xs_hbm, src_hbm, dst_hbm, out_hbm, acc, buf, src_v, dst_v, sem):
    c = lax.axis_index("c")
    s = lax.axis_index("s")
    base = s * RPT
    for k in range(RPT // 128):
        sl = pl.ds(base + k * 128, 128)
        pltpu.sync_copy(xs_hbm.at[sl], buf)
        pltpu.sync_copy(buf, acc.at[sl])
    pltpu.sync_copy(src_hbm.at[c, s], src_v)
    pltpu.sync_copy(dst_hbm.at[c, s], dst_v)
    plsc.subcore_barrier()

    def body(j, carry):
        pltpu.async_copy(xs_hbm.at[src_v.at[j]], buf, sem).wait()
        pltpu.sync_copy(buf, acc.at[dst_v.at[j]], add=True)
        return carry

    lax.fori_loop(0, NB, body, 0)
    plsc.subcore_barrier()
    for k in range(RPT // 128):
        sl = pl.ds(base + k * 128, 128)
        pltpu.sync_copy(acc.at[sl], buf)
        pltpu.sync_copy(buf, out_hbm.at[c, sl])


_ROWS = 256
_GRID = NPAD // _ROWS


def _prep1_body(degp_ref, x_ref, w_ref, xs_ref, dinv_ref):
    deg = degp_ref[0, :, 0] + degp_ref[1, :, 0] - 1.0
    dinv = lax.rsqrt(deg)
    xw = jnp.dot(x_ref[...], w_ref[...], preferred_element_type=jnp.float32)
    xs_ref[...] = xw * dinv[:, None]
    dinv_ref[...] = jnp.broadcast_to(dinv[:, None], dinv_ref.shape)


def _mid_body(dinv_ref, p_ref, xs_ref, b_ref, w_ref, out_ref):
    dinv = dinv_ref[:, 0]
    tot = p_ref[0] + p_ref[1] - xs_ref[...]
    h = jnp.maximum(tot * dinv[:, None] + b_ref[...], 0.0)
    out_ref[...] = jnp.dot(h, w_ref[...], preferred_element_type=jnp.float32) * dinv[:, None]


def _final_body(dinv_ref, p_ref, xs_ref, b_ref, out_ref):
    dinv = dinv_ref[:, 0]
    tot = p_ref[0] + p_ref[1] - xs_ref[...]
    out_ref[...] = tot * dinv[:, None] + b_ref[...]


_rows_spec = pl.BlockSpec((_ROWS, D), lambda i: (i, 0))
_dinv_spec = pl.BlockSpec((_ROWS, 16), lambda i: (i, 0))
_part_spec = pl.BlockSpec((NC, _ROWS, D), lambda i: (0, i, 0))
_w_spec = pl.BlockSpec((D, D), lambda i: (0, 0))
_b_spec = pl.BlockSpec((1, D), lambda i: (0, 0))
_f32 = lambda shape: jax.ShapeDtypeStruct(shape, jnp.float32)

_prep1 = pl.pallas_call(
    _prep1_body,
    grid=(_GRID,),
    in_specs=[_part_spec, _rows_spec, _w_spec],
    out_specs=[_rows_spec, _dinv_spec],
    out_shape=[_f32((NPAD, D)), _f32((NPAD, 16))],
)

_mid = pl.pallas_call(
    _mid_body,
    grid=(_GRID,),
    in_specs=[_dinv_spec, _part_spec, _rows_spec, _b_spec, _w_spec],
    out_specs=_rows_spec,
    out_shape=_f32((NPAD, D)),
)

_final = pl.pallas_call(
    _final_body,
    grid=(_GRID,),
    in_specs=[_dinv_spec, _part_spec, _rows_spec, _b_spec],
    out_specs=_rows_spec,
    out_shape=_f32((NPAD, D)),
)


@jax.jit
def kernel(x, edge_index, W1, b1, W2, b2):
    ei = edge_index.astype(jnp.int32)
    npadrows = NPAD - N_NODES
    pad_idx = N_NODES + (jnp.arange(EPAD - N_EDGES, dtype=jnp.int32) % npadrows)
    src = jnp.concatenate([ei[0], pad_idx]).reshape(NC, NS, NB, BLK)
    dst = jnp.concatenate([ei[1], pad_idx]).reshape(NC, NS, NB, BLK)
    x_pad = jnp.pad(x, ((0, npadrows), (0, 0)))
    ones_tab = jnp.ones((NPAD, D), jnp.float32)

    degp = _scatter_kernel(ones_tab, dst, dst)
    xs1, dinv = _prep1(degp, x_pad, W1)
    p1 = _scatter_kernel(xs1, src, dst)
    xs2 = _mid(dinv, p1, xs1, b1.reshape(1, D), W2)
    p2 = _scatter_kernel(xs2, src, dst)
    out = _final(dinv, p2, xs2, b2.reshape(1, D))
    return out[:N_NODES]


# speedup vs baseline: 21.4171x; 1.3874x over previous
import functools

import jax
import jax.numpy as jnp
from jax import lax
from jax.experimental import pallas as pl
from jax.experimental.pallas import tpu as pltpu
from jax.experimental.pallas import tpu_sc as plsc

N_NODES = 10000
D = 128
N_EDGES = 320000

NC = 2
NS = 16
NW = NC * NS

RPT = 640
NPAD = NS * RPT
BLK = 128
NB = 80
NBC = 16
EPT = NB * BLK
EPAD = NW * EPT

_mesh = plsc.VectorSubcoreMesh(core_axis_name="c", subcore_axis_name="s")


@functools.partial(
    pl.kernel,
    mesh=_mesh,
    out_type=jax.ShapeDtypeStruct((NC, NPAD, D), jnp.float32),
    scratch_types=[
        pltpu.VMEM_SHARED((NPAD, D), jnp.float32),
        pltpu.VMEM((2, BLK, D), jnp.float32),
        pltpu.VMEM((NBC, BLK), jnp.int32),
        pltpu.VMEM((NBC, BLK), jnp.int32),
        pltpu.SemaphoreType.DMA,
        pltpu.SemaphoreType.DMA,
    ],
)
def _scatter_kernel(xs_hbm, src_hbm, dst_hbm, out_hbm, acc, buf, src_v, dst_v, sem0, sem1):
    c = lax.axis_index("c")
    s = lax.axis_index("s")
    base = s * RPT
    for k in range(RPT // 128):
        sl = pl.ds(base + k * 128, 128)
        pltpu.sync_copy(xs_hbm.at[sl], buf.at[0])
        pltpu.sync_copy(buf.at[0], acc.at[sl])
    plsc.subcore_barrier()

    sems = (sem0, sem1)

    @pl.loop(0, NB, step=NBC)
    def _chunk(j0):
        pltpu.sync_copy(src_hbm.at[c, s, pl.ds(j0, NBC)], src_v)
        pltpu.sync_copy(dst_hbm.at[c, s, pl.ds(j0, NBC)], dst_v)
        pltpu.async_copy(xs_hbm.at[src_v.at[0]], buf.at[0], sem0)
        for i in range(NBC - 1):
            pltpu.make_async_copy(xs_hbm.at[src_v.at[i]], buf.at[i % 2], sems[i % 2]).wait()
            pltpu.async_copy(xs_hbm.at[src_v.at[i + 1]], buf.at[(i + 1) % 2], sems[(i + 1) % 2])
            pltpu.sync_copy(buf.at[i % 2], acc.at[dst_v.at[i]], add=True)
        i = NBC - 1
        pltpu.make_async_copy(xs_hbm.at[src_v.at[i]], buf.at[i % 2], sems[i % 2]).wait()
        pltpu.sync_copy(buf.at[i % 2], acc.at[dst_v.at[i]], add=True)

    plsc.subcore_barrier()
    for k in range(RPT // 128):
        sl = pl.ds(base + k * 128, 128)
        pltpu.sync_copy(acc.at[sl], buf.at[0])
        pltpu.sync_copy(buf.at[0], out_hbm.at[c, sl])


@functools.partial(
    pl.kernel,
    mesh=_mesh,
    out_type=jax.ShapeDtypeStruct((NC, NPAD, D), jnp.float32),
    scratch_types=[
        pltpu.VMEM_SHARED((NPAD, D), jnp.float32),
        pltpu.VMEM((BLK, D), jnp.float32),
        pltpu.VMEM((NB, BLK), jnp.int32),
    ],
)
def _deg_kernel(ones_hbm, dst_hbm, out_hbm, acc, buf, dst_v):
    c = lax.axis_index("c")
    s = lax.axis_index("s")
    base = s * RPT
    pltpu.sync_copy(ones_hbm, buf)
    for k in range(RPT // 128):
        pltpu.sync_copy(buf, acc.at[pl.ds(base + k * 128, 128)])
    pltpu.sync_copy(dst_hbm.at[c, s], dst_v)
    plsc.subcore_barrier()

    @pl.loop(0, NB)
    def _edges(j):
        pltpu.sync_copy(buf, acc.at[dst_v.at[j]], add=True)

    plsc.subcore_barrier()
    for k in range(RPT // 128):
        sl = pl.ds(base + k * 128, 128)
        pltpu.sync_copy(acc.at[sl], buf)
        pltpu.sync_copy(buf, out_hbm.at[c, sl])


_ROWS = 256
_GRID = NPAD // _ROWS


def _prep1_body(degp_ref, x_ref, w_ref, xs_ref, dinv_ref):
    deg = degp_ref[0, :, 0] + degp_ref[1, :, 0] - 1.0
    dinv = lax.rsqrt(deg)
    xw = jnp.dot(x_ref[...], w_ref[...], preferred_element_type=jnp.float32)
    xs_ref[...] = xw * dinv[:, None]
    dinv_ref[...] = jnp.broadcast_to(dinv[:, None], dinv_ref.shape)


def _mid_body(dinv_ref, p_ref, xs_ref, b_ref, w_ref, out_ref):
    dinv = dinv_ref[:, 0]
    tot = p_ref[0] + p_ref[1] - xs_ref[...]
    h = jnp.maximum(tot * dinv[:, None] + b_ref[...], 0.0)
    out_ref[...] = jnp.dot(h, w_ref[...], preferred_element_type=jnp.float32) * dinv[:, None]


def _final_body(dinv_ref, p_ref, xs_ref, b_ref, out_ref):
    dinv = dinv_ref[:, 0]
    tot = p_ref[0] + p_ref[1] - xs_ref[...]
    out_ref[...] = tot * dinv[:, None] + b_ref[...]


_rows_spec = pl.BlockSpec((_ROWS, D), lambda i: (i, 0))
_dinv_spec = pl.BlockSpec((_ROWS, 16), lambda i: (i, 0))
_part_spec = pl.BlockSpec((NC, _ROWS, D), lambda i: (0, i, 0))
_w_spec = pl.BlockSpec((D, D), lambda i: (0, 0))
_b_spec = pl.BlockSpec((1, D), lambda i: (0, 0))
_f32 = lambda shape: jax.ShapeDtypeStruct(shape, jnp.float32)

_prep1 = pl.pallas_call(
    _prep1_body,
    grid=(_GRID,),
    in_specs=[_part_spec, _rows_spec, _w_spec],
    out_specs=[_rows_spec, _dinv_spec],
    out_shape=[_f32((NPAD, D)), _f32((NPAD, 16))],
)

_mid = pl.pallas_call(
    _mid_body,
    grid=(_GRID,),
    in_specs=[_dinv_spec, _part_spec, _rows_spec, _b_spec, _w_spec],
    out_specs=_rows_spec,
    out_shape=_f32((NPAD, D)),
)

_final = pl.pallas_call(
    _final_body,
    grid=(_GRID,),
    in_specs=[_dinv_spec, _part_spec, _rows_spec, _b_spec],
    out_specs=_rows_spec,
    out_shape=_f32((NPAD, D)),
)


@jax.jit
def kernel(x, edge_index, W1, b1, W2, b2):
    ei = edge_index.astype(jnp.int32)
    npadrows = NPAD - N_NODES
    pad_idx = N_NODES + (jnp.arange(EPAD - N_EDGES, dtype=jnp.int32) % npadrows)
    src = jnp.concatenate([ei[0], pad_idx]).reshape(NC, NS, NB, BLK)
    dst = jnp.concatenate([ei[1], pad_idx]).reshape(NC, NS, NB, BLK)
    x_pad = jnp.pad(x, ((0, npadrows), (0, 0)))
    ones_tab = jnp.ones((BLK, D), jnp.float32)

    degp = _deg_kernel(ones_tab, dst)
    xs1, dinv = _prep1(degp, x_pad, W1)
    p1 = _scatter_kernel(xs1, src, dst)
    xs2 = _mid(dinv, p1, xs1, b1.reshape(1, D), W2)
    p2 = _scatter_kernel(xs2, src, dst)
    out = _final(dinv, p2, xs2, b2.reshape(1, D))
    return out[:N_NODES]


# speedup vs baseline: 22.8752x; 1.0681x over previous
import functools

import jax
import jax.numpy as jnp
from jax import lax
from jax.experimental import pallas as pl
from jax.experimental.pallas import tpu as pltpu
from jax.experimental.pallas import tpu_sc as plsc

N_NODES = 10000
D = 128
N_EDGES = 320000

NC = 2
NS = 16
NW = NC * NS

RPT = 640
NPAD = NS * RPT
BLK = 64
NB = 160
NBC = 16
NBUF = 4
DBLK = 128
EPT = NB * BLK
EPAD = NW * EPT

_mesh = plsc.VectorSubcoreMesh(core_axis_name="c", subcore_axis_name="s")


@functools.partial(
    pl.kernel,
    mesh=_mesh,
    out_type=jax.ShapeDtypeStruct((NC, NPAD, D), jnp.float32),
    scratch_types=[
        pltpu.VMEM_SHARED((NPAD, D), jnp.float32),
        pltpu.VMEM((NBUF, BLK, D), jnp.float32),
        pltpu.VMEM((NBC, BLK), jnp.int32),
        pltpu.VMEM((NBC, BLK), jnp.int32),
        [pltpu.SemaphoreType.DMA] * NBUF,
    ],
)
def _scatter_kernel(xs_hbm, src_hbm, dst_hbm, out_hbm, acc, buf, src_v, dst_v, sems):
    c = lax.axis_index("c")
    s = lax.axis_index("s")
    base = s * RPT
    for k in range(RPT // BLK):
        sl = pl.ds(base + k * BLK, BLK)
        pltpu.sync_copy(xs_hbm.at[sl], buf.at[0])
        pltpu.sync_copy(buf.at[0], acc.at[sl])
    plsc.subcore_barrier()

    @pl.loop(0, NB, step=NBC)
    def _chunk(j0):
        pltpu.sync_copy(src_hbm.at[c, s, pl.ds(j0, NBC)], src_v)
        pltpu.sync_copy(dst_hbm.at[c, s, pl.ds(j0, NBC)], dst_v)
        for b in range(NBUF - 1):
            pltpu.async_copy(xs_hbm.at[src_v.at[b]], buf.at[b], sems[b])
        for i in range(NBC):
            bi = i % NBUF
            pltpu.make_async_copy(xs_hbm.at[src_v.at[i]], buf.at[bi], sems[bi]).wait()
            if i + NBUF - 1 < NBC:
                bn = (i + NBUF - 1) % NBUF
                pltpu.async_copy(xs_hbm.at[src_v.at[i + NBUF - 1]], buf.at[bn], sems[bn])
            pltpu.sync_copy(buf.at[bi], acc.at[dst_v.at[i]], add=True)

    plsc.subcore_barrier()
    for k in range(RPT // BLK):
        sl = pl.ds(base + k * BLK, BLK)
        pltpu.sync_copy(acc.at[sl], buf.at[0])
        pltpu.sync_copy(buf.at[0], out_hbm.at[c, sl])


DNB = EPT // DBLK


@functools.partial(
    pl.kernel,
    mesh=_mesh,
    out_type=jax.ShapeDtypeStruct((NC, NPAD, D), jnp.float32),
    scratch_types=[
        pltpu.VMEM_SHARED((NPAD, D), jnp.float32),
        pltpu.VMEM((DBLK, D), jnp.float32),
        pltpu.VMEM((DNB, DBLK), jnp.int32),
    ],
)
def _deg_kernel(ones_hbm, dst_hbm, out_hbm, acc, buf, dst_v):
    c = lax.axis_index("c")
    s = lax.axis_index("s")
    base = s * RPT
    pltpu.sync_copy(ones_hbm, buf)
    for k in range(RPT // DBLK):
        pltpu.sync_copy(buf, acc.at[pl.ds(base + k * DBLK, DBLK)])
    pltpu.sync_copy(dst_hbm.at[c, s], dst_v)
    plsc.subcore_barrier()

    @pl.loop(0, DNB)
    def _edges(j):
        pltpu.sync_copy(buf, acc.at[dst_v.at[j]], add=True)

    plsc.subcore_barrier()
    for k in range(RPT // DBLK):
        sl = pl.ds(base + k * DBLK, DBLK)
        pltpu.sync_copy(acc.at[sl], buf)
        pltpu.sync_copy(buf, out_hbm.at[c, sl])


_ROWS = 256
_GRID = NPAD // _ROWS


def _prep1_body(degp_ref, x_ref, w_ref, xs_ref, dinv_ref):
    deg = degp_ref[0, :, 0] + degp_ref[1, :, 0] - 1.0
    dinv = lax.rsqrt(deg)
    xw = jnp.dot(x_ref[...], w_ref[...], preferred_element_type=jnp.float32)
    xs_ref[...] = xw * dinv[:, None]
    dinv_ref[...] = jnp.broadcast_to(dinv[:, None], dinv_ref.shape)


def _mid_body(dinv_ref, p_ref, xs_ref, b_ref, w_ref, out_ref):
    dinv = dinv_ref[:, 0]
    tot = p_ref[0] + p_ref[1] - xs_ref[...]
    h = jnp.maximum(tot * dinv[:, None] + b_ref[...], 0.0)
    out_ref[...] = jnp.dot(h, w_ref[...], preferred_element_type=jnp.float32) * dinv[:, None]


def _final_body(dinv_ref, p_ref, xs_ref, b_ref, out_ref):
    dinv = dinv_ref[:, 0]
    tot = p_ref[0] + p_ref[1] - xs_ref[...]
    out_ref[...] = tot * dinv[:, None] + b_ref[...]


_rows_spec = pl.BlockSpec((_ROWS, D), lambda i: (i, 0))
_dinv_spec = pl.BlockSpec((_ROWS, 16), lambda i: (i, 0))
_part_spec = pl.BlockSpec((NC, _ROWS, D), lambda i: (0, i, 0))
_w_spec = pl.BlockSpec((D, D), lambda i: (0, 0))
_b_spec = pl.BlockSpec((1, D), lambda i: (0, 0))
_f32 = lambda shape: jax.ShapeDtypeStruct(shape, jnp.float32)

_prep1 = pl.pallas_call(
    _prep1_body,
    grid=(_GRID,),
    in_specs=[_part_spec, _rows_spec, _w_spec],
    out_specs=[_rows_spec, _dinv_spec],
    out_shape=[_f32((NPAD, D)), _f32((NPAD, 16))],
)

_mid = pl.pallas_call(
    _mid_body,
    grid=(_GRID,),
    in_specs=[_dinv_spec, _part_spec, _rows_spec, _b_spec, _w_spec],
    out_specs=_rows_spec,
    out_shape=_f32((NPAD, D)),
)

_final = pl.pallas_call(
    _final_body,
    grid=(_GRID,),
    in_specs=[_dinv_spec, _part_spec, _rows_spec, _b_spec],
    out_specs=_rows_spec,
    out_shape=_f32((NPAD, D)),
)


@jax.jit
def kernel(x, edge_index, W1, b1, W2, b2):
    ei = edge_index.astype(jnp.int32)
    npadrows = NPAD - N_NODES
    pad_idx = N_NODES + (jnp.arange(EPAD - N_EDGES, dtype=jnp.int32) % npadrows)
    src = jnp.concatenate([ei[0], pad_idx]).reshape(NC, NS, NB, BLK)
    dst = jnp.concatenate([ei[1], pad_idx]).reshape(NC, NS, NB, BLK)
    dst_d = dst.reshape(NC, NS, DNB, DBLK)
    x_pad = jnp.pad(x, ((0, npadrows), (0, 0)))
    ones_tab = jnp.ones((DBLK, D), jnp.float32)

    degp = _deg_kernel(ones_tab, dst_d)
    xs1, dinv = _prep1(degp, x_pad, W1)
    p1 = _scatter_kernel(xs1, src, dst)
    xs2 = _mid(dinv, p1, xs1, b1.reshape(1, D), W2)
    p2 = _scatter_kernel(xs2, src, dst)
    out = _final(dinv, p2, xs2, b2.reshape(1, D))
    return out[:N_NODES]


# speedup vs baseline: 25.5443x; 1.1167x over previous
import functools

import jax
import jax.numpy as jnp
from jax import lax
from jax.experimental import pallas as pl
from jax.experimental.pallas import tpu as pltpu
from jax.experimental.pallas import tpu_sc as plsc

N_NODES = 10000
D = 128
N_EDGES = 320000

NC = 2
NS = 16
NW = NC * NS

RPT = 640
NPAD = NS * RPT
BLK = 64
NB = 160
NBC = 16
NBUF = 4
DBLK = 128
EPT = NB * BLK
EPAD = NW * EPT

_mesh = plsc.VectorSubcoreMesh(core_axis_name="c", subcore_axis_name="s")


@functools.partial(
    pl.kernel,
    mesh=_mesh,
    out_type=jax.ShapeDtypeStruct((NC, NPAD, D), jnp.float32),
    scratch_types=[
        pltpu.VMEM_SHARED((NPAD, D), jnp.float32),
        pltpu.VMEM((NBUF, BLK, D), jnp.float32),
        pltpu.VMEM((NBC, BLK), jnp.int32),
        pltpu.VMEM((NBC, BLK), jnp.int32),
        [pltpu.SemaphoreType.DMA] * NBUF,
    ],
)
def _scatter_kernel(xs_hbm, src_hbm, dst_hbm, out_hbm, acc, buf, src_v, dst_v, sems):
    c = lax.axis_index("c")
    s = lax.axis_index("s")
    base = s * RPT
    for k in range(RPT // BLK):
        sl = pl.ds(base + k * BLK, BLK)
        pltpu.sync_copy(xs_hbm.at[sl], buf.at[0])
        pltpu.sync_copy(buf.at[0], acc.at[sl])
    plsc.subcore_barrier()

    @pl.loop(0, NB, step=NBC)
    def _chunk(j0):
        pltpu.sync_copy(src_hbm.at[c, s, pl.ds(j0, NBC)], src_v)
        pltpu.sync_copy(dst_hbm.at[c, s, pl.ds(j0, NBC)], dst_v)
        for b in range(NBUF - 1):
            pltpu.async_copy(xs_hbm.at[src_v.at[b]], buf.at[b], sems[b])
        for i in range(NBC):
            bi = i % NBUF
            pltpu.make_async_copy(xs_hbm.at[src_v.at[i]], buf.at[bi], sems[bi]).wait()
            if i + NBUF - 1 < NBC:
                bn = (i + NBUF - 1) % NBUF
                pltpu.async_copy(xs_hbm.at[src_v.at[i + NBUF - 1]], buf.at[bn], sems[bn])
            pltpu.sync_copy(buf.at[bi], acc.at[dst_v.at[i]], add=True)

    plsc.subcore_barrier()
    for k in range(RPT // BLK):
        sl = pl.ds(base + k * BLK, BLK)
        pltpu.sync_copy(acc.at[sl], buf.at[0])
        pltpu.sync_copy(buf.at[0], out_hbm.at[c, sl])


DNB = EPT // DBLK


@functools.partial(
    pl.kernel,
    mesh=_mesh,
    out_type=jax.ShapeDtypeStruct((NC, NPAD, D), jnp.float32),
    scratch_types=[
        pltpu.VMEM_SHARED((NPAD, D), jnp.float32),
        pltpu.VMEM((DBLK, D), jnp.float32),
        pltpu.VMEM((DNB, DBLK), jnp.int32),
    ],
)
def _deg_kernel(ones_hbm, dst_hbm, out_hbm, acc, buf, dst_v):
    c = lax.axis_index("c")
    s = lax.axis_index("s")
    base = s * RPT
    pltpu.sync_copy(ones_hbm, buf)
    for k in range(RPT // DBLK):
        pltpu.sync_copy(buf, acc.at[pl.ds(base + k * DBLK, DBLK)])
    pltpu.sync_copy(dst_hbm.at[c, s], dst_v)
    plsc.subcore_barrier()

    @pl.loop(0, DNB)
    def _edges(j):
        pltpu.sync_copy(buf, acc.at[dst_v.at[j]], add=True)

    plsc.subcore_barrier()
    for k in range(RPT // DBLK):
        sl = pl.ds(base + k * DBLK, DBLK)
        pltpu.sync_copy(acc.at[sl], buf)
        pltpu.sync_copy(buf, out_hbm.at[c, sl])


_ROWS = 1024
_GRID = NPAD // _ROWS


def _prep0_body(x_ref, w_ref, xw_ref):
    xw_ref[...] = jnp.dot(x_ref[...], w_ref[...], preferred_element_type=jnp.float32)


def _prep1_body(degp_ref, xw_ref, xs_ref, dinv_ref):
    deg = degp_ref[0, :, 0] + degp_ref[1, :, 0] - 1.0
    dinv = lax.rsqrt(deg)
    xs_ref[...] = xw_ref[...] * dinv[:, None]
    dinv_ref[...] = jnp.broadcast_to(dinv[:, None], dinv_ref.shape)


def _mid_body(dinv_ref, p_ref, xs_ref, b_ref, w_ref, out_ref):
    dinv = dinv_ref[:, 0]
    tot = p_ref[0] + p_ref[1] - xs_ref[...]
    h = jnp.maximum(tot * dinv[:, None] + b_ref[...], 0.0)
    out_ref[...] = jnp.dot(h, w_ref[...], preferred_element_type=jnp.float32) * dinv[:, None]


def _final_body(dinv_ref, p_ref, xs_ref, b_ref, out_ref):
    dinv = dinv_ref[:, 0]
    tot = p_ref[0] + p_ref[1] - xs_ref[...]
    out_ref[...] = tot * dinv[:, None] + b_ref[...]


_rows_spec = pl.BlockSpec((_ROWS, D), lambda i: (i, 0))
_dinv_spec = pl.BlockSpec((_ROWS, 16), lambda i: (i, 0))
_part_spec = pl.BlockSpec((NC, _ROWS, D), lambda i: (0, i, 0))
_degp_spec = pl.BlockSpec((NC, _ROWS, D), lambda i: (0, i, 0))
_w_spec = pl.BlockSpec((D, D), lambda i: (0, 0))
_b_spec = pl.BlockSpec((1, D), lambda i: (0, 0))
_f32 = lambda shape: jax.ShapeDtypeStruct(shape, jnp.float32)

_prep0 = pl.pallas_call(
    _prep0_body,
    grid=(_GRID,),
    in_specs=[_rows_spec, _w_spec],
    out_specs=_rows_spec,
    out_shape=_f32((NPAD, D)),
)

_prep1 = pl.pallas_call(
    _prep1_body,
    grid=(_GRID,),
    in_specs=[_degp_spec, _rows_spec],
    out_specs=[_rows_spec, _dinv_spec],
    out_shape=[_f32((NPAD, D)), _f32((NPAD, 16))],
)

_mid = pl.pallas_call(
    _mid_body,
    grid=(_GRID,),
    in_specs=[_dinv_spec, _part_spec, _rows_spec, _b_spec, _w_spec],
    out_specs=_rows_spec,
    out_shape=_f32((NPAD, D)),
)

_final = pl.pallas_call(
    _final_body,
    grid=(_GRID,),
    in_specs=[_dinv_spec, _part_spec, _rows_spec, _b_spec],
    out_specs=_rows_spec,
    out_shape=_f32((NPAD, D)),
)


@jax.jit
def kernel(x, edge_index, W1, b1, W2, b2):
    ei = edge_index.astype(jnp.int32)
    npadrows = NPAD - N_NODES
    pad_idx = N_NODES + (jnp.arange(EPAD - N_EDGES, dtype=jnp.int32) % npadrows)
    src = jnp.concatenate([ei[0], pad_idx]).reshape(NC, NS, NB, BLK)
    dst = jnp.concatenate([ei[1], pad_idx]).reshape(NC, NS, NB, BLK)
    dst_d = dst.reshape(NC, NS, DNB, DBLK)
    x_pad = jnp.pad(x, ((0, npadrows), (0, 0)))
    ones_tab = jnp.ones((DBLK, D), jnp.float32)

    xw1 = _prep0(x_pad, W1)
    degp = _deg_kernel(ones_tab, dst_d)
    xs1, dinv = _prep1(degp, xw1)
    p1 = _scatter_kernel(xs1, src, dst)
    xs2 = _mid(dinv, p1, xs1, b1.reshape(1, D), W2)
    p2 = _scatter_kernel(xs2, src, dst)
    out = _final(dinv, p2, xs2, b2.reshape(1, D))
    return out[:N_NODES]
